# baseline (speedup 1.0000x reference)
import functools

import jax
import jax.numpy as jnp
from jax import lax
from jax.experimental import pallas as pl
from jax.experimental.pallas import tpu as pltpu
from jax.experimental.pallas import tpu_sc as plsc

_BATCH = 16384
_D = 64
_N_TBL = 1000000
_NC = 2
_NS = 16
_NW = _NC * _NS
_BPW = _BATCH // _NW
_CHUNK = 128
_NCHUNK = _BPW // _CHUNK
_L = 16
_HALF_ROWS = _BPW // 2


def _lane_bcast(v, r):
    return v.at[jnp.full((_L,), r, jnp.int32)].get(
        mode=lax.GatherScatterMode.PROMISE_IN_BOUNDS)


def _sc_body(users_h, pos_h, neg_h, ut_h, it_h,
             xp_out, reg_out,
             idx_u, idx_p, idx_n, par_u, par_p, par_n,
             ru, rp, rn, parts, racc, sem):
    wid = lax.axis_index("s") * _NC + lax.axis_index("c")
    base = wid * _BPW

    for j in range(_NCHUNK):
        sl = pl.ds(base + j * _CHUNK, _CHUNK)
        pltpu.sync_copy(users_h.at[sl], idx_u.at[j])
        pltpu.sync_copy(pos_h.at[sl], idx_p.at[j])
        pltpu.sync_copy(neg_h.at[sl], idx_n.at[j])

    one = jnp.full((_L,), 1, jnp.int32)
    for j in range(_NCHUNK):
        for t in range(_CHUNK // _L):
            s = pl.ds(t * _L, _L)
            ds_flat = pl.ds(j * _CHUNK + t * _L, _L)
            v = idx_u[j, s]
            d2 = jnp.right_shift(v, 13)
            m = jnp.bitwise_and(v, 8191)
            h = jnp.right_shift(m, 12)
            par_u[ds_flat] = h.astype(jnp.float32)
            idx_u[j, s] = d2 * 4096 + jnp.bitwise_and(m, 4095)
            for idx, par in ((idx_p, par_p), (idx_n, par_n)):
                v = idx[j, s]
                par[ds_flat] = jnp.bitwise_and(v, one).astype(jnp.float32)
                idx[j, s] = jnp.right_shift(v, one)

    zero = jnp.zeros((_L,), jnp.float32)
    sacc = zero

    for half in range(2):
        copies = []
        for j in range(_HALF_ROWS // _CHUNK):
            jj = half * (_HALF_ROWS // _CHUNK) + j
            dst = pl.ds(j * _CHUNK, _CHUNK)
            copies.append(pltpu.async_copy(ut_h.at[idx_u.at[jj]], ru.at[dst], sem))
            copies.append(pltpu.async_copy(it_h.at[idx_p.at[jj]], rp.at[dst], sem))
            copies.append(pltpu.async_copy(it_h.at[idx_n.at[jj]], rn.at[dst], sem))
        for c in copies:
            c.wait()

        def grp_body(g, sacc):
            gbase = half * _HALF_ROWS + g * _L
            pu16 = par_u[pl.ds(gbase, _L)]
            pp16 = par_p[pl.ds(gbase, _L)]
            pn16 = par_n[pl.ds(gbase, _L)]
            for r in range(_L):
                i = g * _L + r
                fu = _lane_bcast(pu16, r)
                fp = _lane_bcast(pp16, r)
                fn = _lane_bcast(pn16, r)
                pv = zero
                for kk in range(_D // _L):
                    lo = pl.ds(kk * _L, _L)
                    hi = pl.ds(_D + kk * _L, _L)
                    ul, uh = ru[i, lo], ru[i, hi]
                    pl_, ph = rp[i, lo], rp[i, hi]
                    nl, nh = rn[i, lo], rn[i, hi]
                    u = ul + fu * (uh - ul)
                    p = pl_ + fp * (ph - pl_)
                    n = nl + fn * (nh - nl)
                    pv = pv + u * (p - n)
                    sacc = sacc + u * u + p * p + n * n
                parts[pl.ds((gbase + r) * _L, _L)] = pv
            return sacc

        sacc = lax.fori_loop(0, _HALF_ROWS // _L, grp_body, sacc)

    racc[...] = sacc
    pltpu.sync_copy(parts, xp_out.at[pl.ds(base * _L, _BPW * _L)])
    pltpu.sync_copy(racc, reg_out.at[pl.ds(wid * _L, _L)])


_sc_gather_dot = functools.partial(
    pl.kernel,
    mesh=plsc.VectorSubcoreMesh(core_axis_name="c", subcore_axis_name="s"),
    out_type=[
        jax.ShapeDtypeStruct((_BATCH * _L,), jnp.float32),
        jax.ShapeDtypeStruct((_NW * _L,), jnp.float32),
    ],
    scratch_types=[
        pltpu.VMEM((_NCHUNK, _CHUNK), jnp.int32),
        pltpu.VMEM((_NCHUNK, _CHUNK), jnp.int32),
        pltpu.VMEM((_NCHUNK, _CHUNK), jnp.int32),
        pltpu.VMEM((_BPW,), jnp.float32),
        pltpu.VMEM((_BPW,), jnp.float32),
        pltpu.VMEM((_BPW,), jnp.float32),
        pltpu.VMEM((_HALF_ROWS, 2 * _D), jnp.float32),
        pltpu.VMEM((_HALF_ROWS, 2 * _D), jnp.float32),
        pltpu.VMEM((_HALF_ROWS, 2 * _D), jnp.float32),
        pltpu.VMEM((_BPW * _L,), jnp.float32),
        pltpu.VMEM((_L,), jnp.float32),
        pltpu.SemaphoreType.DMA,
    ],
)(_sc_body)


_TS = 4096
_NSB = pl.cdiv(_N_TBL, 2 * _TS)


def _transpose_body(lo_ref, hi_ref, out_ref):
    out_ref[:, : _D] = lo_ref[...].T
    out_ref[:, _D :] = hi_ref[...].T


_transpose = pl.pallas_call(
    _transpose_body,
    grid=(_NSB,),
    in_specs=[
        pl.BlockSpec((_D, _TS), lambda g: (0, 2 * g)),
        pl.BlockSpec(
            (_D, _TS),
            lambda g: (0, jnp.minimum(2 * g + 1, _N_TBL // _TS - 1)),
        ),
    ],
    out_specs=pl.BlockSpec((_TS, 2 * _D), lambda g: (g, 0)),
    out_shape=jax.ShapeDtypeStruct((_NSB * _TS, 2 * _D), jnp.float32),
)


def _finish_body(xp_ref, regp_ref, loss_ref, reg_ref):
    xp = xp_ref[...]
    grp = lax.broadcasted_iota(jnp.int32, (128, 8), 0) // _L
    col = lax.broadcasted_iota(jnp.int32, (128, 8), 1)
    diff = (grp - col).astype(jnp.float32)
    sel = 1.0 - jnp.abs(jnp.sign(diff))
    x = lax.dot_general(xp, sel, (((1,), (0,)), ((), ())),
                        preferred_element_type=jnp.float32)
    ls = jnp.minimum(x, 0.0) - jnp.log1p(jnp.exp(-jnp.abs(x)))
    loss_ref[...] = jnp.reshape(-jnp.sum(ls) * (1.0 / _BATCH), (1, 1))
    reg_ref[...] = jnp.reshape(jnp.sum(regp_ref[...]) * (1.0 / _BATCH), (1, 1))


_finish = pl.pallas_call(
    _finish_body,
    out_shape=(
        jax.ShapeDtypeStruct((1, 1), jnp.float32),
        jax.ShapeDtypeStruct((1, 1), jnp.float32),
    ),
)


def kernel(users, pos, neg, user_table, item_table):
    ut_t = user_table.T
    ut2 = _transpose(ut_t, ut_t)
    it2 = item_table.reshape(-1, 2 * _D)
    xp, regp = _sc_gather_dot(users, pos, neg, ut2, it2)
    loss, reg = _finish(xp.reshape(_BATCH // 8, 128), regp.reshape(4, 128))
    return loss.reshape(()), reg.reshape(())


# speedup vs baseline: 1.7744x; 1.7744x over previous
import functools

import jax
import jax.numpy as jnp
from jax import lax
from jax.experimental import pallas as pl
from jax.experimental.pallas import tpu as pltpu
from jax.experimental.pallas import tpu_sc as plsc

_BATCH = 16384
_D = 64
_N_TBL = 1000000
_NC = 2
_NS = 16
_NW = _NC * _NS
_BPW = _BATCH // _NW
_CHUNK = 128
_NCHUNK = _BPW // _CHUNK
_L = 16
_TS = 4096


def _lane_bcast(v, r):
    return v.at[jnp.full((_L,), r, jnp.int32)].get(
        mode=lax.GatherScatterMode.PROMISE_IN_BOUNDS)


_CH = 32
_NRND = _BPW // _CH
_TPB = 8


def _sc_body(users_h, pos_h, neg_h, ut_h, it_h,
             xp_out, reg_out,
             idx_u, par_u, sdx_p, sdx_n,
             ru, rp, rn, parts, racc, sem):
    wid = lax.axis_index("s") * _NC + lax.axis_index("c")
    base = wid * _BPW

    for j in range(_NCHUNK):
        sl = pl.ds(base + j * _CHUNK, _CHUNK)
        pltpu.sync_copy(users_h.at[sl], idx_u.at[j])
        pltpu.sync_copy(pos_h.at[sl], sdx_p.at[j])
        pltpu.sync_copy(neg_h.at[sl], sdx_n.at[j])

    for j in range(_NCHUNK):
        for t in range(_CHUNK // _L):
            s = pl.ds(t * _L, _L)
            v = idx_u[j, s]
            d2 = jnp.right_shift(v, 13)
            m = jnp.bitwise_and(v, 8191)
            h = jnp.right_shift(m, 12)
            par_u[pl.ds(j * _CHUNK + t * _L, _L)] = h.astype(jnp.float32)
            idx_u[j, s] = d2 * _TS + jnp.bitwise_and(m, _TS - 1)

    zero = jnp.zeros((_L,), jnp.float32)

    def round_body(c, sacc):
        jr = lax.shift_right_logical(c, 2)
        orow = jnp.bitwise_and(c, 3) * _CH
        copies = [pltpu.async_copy(
            ut_h.at[idx_u.at[jr, pl.ds(orow, _CH)]], ru, sem)]
        subs = []
        for g in range(_CH // _L):
            goff = orow + g * _L
            vp = sdx_p[jr, pl.ds(goff, _L)]
            vn = sdx_n[jr, pl.ds(goff, _L)]
            for r in range(_L):
                i = g * _L + r
                ep, en = vp[r], vn[r]
                subs.append((jnp.bitwise_and(ep, 7), jnp.bitwise_and(en, 7)))
                copies.append(pltpu.async_copy(
                    it_h.at[lax.shift_right_logical(ep, 3)], rp.at[i], sem))
                copies.append(pltpu.async_copy(
                    it_h.at[lax.shift_right_logical(en, 3)], rn.at[i], sem))
        for cp_ in copies:
            cp_.wait()

        for g in range(_CH // _L):
            pu16 = par_u[pl.ds(c * _CH + g * _L, _L)]
            for r in range(_L):
                i = g * _L + r
                fu = _lane_bcast(pu16, r)
                sp_, sn_ = subs[i]
                pv = zero
                for kk in range(_D // _L):
                    lo = pl.ds(kk * _L, _L)
                    hi = pl.ds(_D + kk * _L, _L)
                    ul, uh = ru[i, lo], ru[i, hi]
                    u = ul + fu * (uh - ul)
                    p = rp[i, sp_, lo]
                    n = rn[i, sn_, lo]
                    pv = pv + u * (p - n)
                    sacc = sacc + u * u + p * p + n * n
                parts[pl.ds((c * _CH + i) * _L, _L)] = pv
        return sacc

    sacc = lax.fori_loop(0, _NRND, round_body, zero)

    racc[...] = sacc
    pltpu.sync_copy(parts, xp_out.at[pl.ds(base * _L, _BPW * _L)])
    pltpu.sync_copy(racc, reg_out.at[pl.ds(wid * _L, _L)])


_sc_gather_dot = functools.partial(
    pl.kernel,
    mesh=plsc.VectorSubcoreMesh(core_axis_name="c", subcore_axis_name="s"),
    out_type=[
        jax.ShapeDtypeStruct((_BATCH * _L,), jnp.float32),
        jax.ShapeDtypeStruct((_NW * _L,), jnp.float32),
    ],
    scratch_types=[
        pltpu.VMEM((_NCHUNK, _CHUNK), jnp.int32),
        pltpu.VMEM((_BPW,), jnp.float32),
        pltpu.VMEM((_NCHUNK, _CHUNK), jnp.int32),
        pltpu.VMEM((_NCHUNK, _CHUNK), jnp.int32),
        pltpu.VMEM((_CH, 2 * _D), jnp.float32),
        pltpu.VMEM((_CH, _TPB, _D), jnp.float32),
        pltpu.VMEM((_CH, _TPB, _D), jnp.float32),
        pltpu.VMEM((_BPW * _L,), jnp.float32),
        pltpu.VMEM((_L,), jnp.float32),
        pltpu.SemaphoreType.DMA,
    ],
)(_sc_body)


_NSB = pl.cdiv(_N_TBL, 2 * _TS)


def _transpose_body(lo_ref, hi_ref, out_ref):
    out_ref[:, : _D] = lo_ref[...].T
    out_ref[:, _D :] = hi_ref[...].T


_transpose = pl.pallas_call(
    _transpose_body,
    grid=(_NSB,),
    in_specs=[
        pl.BlockSpec((_D, _TS), lambda g: (0, 2 * g)),
        pl.BlockSpec(
            (_D, _TS),
            lambda g: (0, jnp.minimum(2 * g + 1, _N_TBL // _TS - 1)),
        ),
    ],
    out_specs=pl.BlockSpec((_TS, 2 * _D), lambda g: (g, 0)),
    out_shape=jax.ShapeDtypeStruct((_NSB * _TS, 2 * _D), jnp.float32),
)


def _finish_body(xp_ref, regp_ref, loss_ref, reg_ref):
    xp = xp_ref[...]
    grp = lax.broadcasted_iota(jnp.int32, (128, 8), 0) // _L
    col = lax.broadcasted_iota(jnp.int32, (128, 8), 1)
    diff = (grp - col).astype(jnp.float32)
    sel = 1.0 - jnp.abs(jnp.sign(diff))
    x = lax.dot_general(xp, sel, (((1,), (0,)), ((), ())),
                        preferred_element_type=jnp.float32)
    ls = jnp.minimum(x, 0.0) - jnp.log1p(jnp.exp(-jnp.abs(x)))
    loss_ref[...] = jnp.reshape(-jnp.sum(ls) * (1.0 / _BATCH), (1, 1))
    reg_ref[...] = jnp.reshape(jnp.sum(regp_ref[...]) * (1.0 / _BATCH), (1, 1))


_finish = pl.pallas_call(
    _finish_body,
    out_shape=(
        jax.ShapeDtypeStruct((1, 1), jnp.float32),
        jax.ShapeDtypeStruct((1, 1), jnp.float32),
    ),
)


def kernel(users, pos, neg, user_table, item_table):
    ut_t = user_table.T
    ut2 = _transpose(ut_t, ut_t)
    it3 = item_table.reshape(-1, _TPB, _D)
    xp, regp = _sc_gather_dot(users, pos, neg, ut2, it3)
    loss, reg = _finish(xp.reshape(_BATCH // 8, 128), regp.reshape(4, 128))
    return loss.reshape(()), reg.reshape(())


# speedup vs baseline: 1.9454x; 1.0964x over previous
import functools

import jax
import jax.numpy as jnp
from jax import lax
from jax.experimental import pallas as pl
from jax.experimental.pallas import tpu as pltpu
from jax.experimental.pallas import tpu_sc as plsc

_BATCH = 16384
_D = 64
_N_TBL = 1000000
_NC = 2
_NS = 16
_NW = _NC * _NS
_BPW = _BATCH // _NW
_CHUNK = 128
_NCHUNK = _BPW // _CHUNK
_L = 16
_TS = 4096


def _lane_bcast(v, r):
    return v.at[jnp.full((_L,), r, jnp.int32)].get(
        mode=lax.GatherScatterMode.PROMISE_IN_BOUNDS)


_CH = 32
_NRND = _BPW // _CH
_TPB = 8


def _sc_body(users_h, pos_h, neg_h, ut_h, it_h,
             xp_out, reg_out,
             idx_u, par_u, sdx_p, sdx_n,
             ru, rp, rn, parts, racc, sem):
    wid = lax.axis_index("s") * _NC + lax.axis_index("c")
    base = wid * _BPW

    for j in range(_NCHUNK):
        sl = pl.ds(base + j * _CHUNK, _CHUNK)
        pltpu.sync_copy(users_h.at[sl], idx_u.at[j])
        pltpu.sync_copy(pos_h.at[sl], sdx_p.at[j])
        pltpu.sync_copy(neg_h.at[sl], sdx_n.at[j])

    for j in range(_NCHUNK):
        for t in range(_CHUNK // _L):
            s = pl.ds(t * _L, _L)
            v = idx_u[j, s]
            d2 = jnp.right_shift(v, 13)
            m = jnp.bitwise_and(v, 8191)
            h = jnp.right_shift(m, 12)
            par_u[pl.ds(j * _CHUNK + t * _L, _L)] = h.astype(jnp.float32)
            idx_u[j, s] = d2 * _TS + jnp.bitwise_and(m, _TS - 1)

    zero = jnp.zeros((_L,), jnp.float32)

    def round_body(c, sacc):
        jr = lax.shift_right_logical(c, 2)
        orow = jnp.bitwise_and(c, 3) * _CH
        copies = [pltpu.async_copy(
            ut_h.at[idx_u.at[jr, pl.ds(orow, _CH)]], ru, sem)]
        for g in range(_CH // _L):
            goff = orow + g * _L
            vp = sdx_p[jr, pl.ds(goff, _L)]
            vn = sdx_n[jr, pl.ds(goff, _L)]
            for r in range(_L):
                i = g * _L + r
                ep, en = vp[r], vn[r]
                copies.append(pltpu.async_copy(
                    it_h.at[lax.shift_right_logical(ep, 3),
                            jnp.bitwise_and(ep, 7)], rp.at[i], sem))
                copies.append(pltpu.async_copy(
                    it_h.at[lax.shift_right_logical(en, 3),
                            jnp.bitwise_and(en, 7)], rn.at[i], sem))
        for cp_ in copies:
            cp_.wait()

        for g in range(_CH // _L):
            pu16 = par_u[pl.ds(c * _CH + g * _L, _L)]
            for r in range(_L):
                i = g * _L + r
                fu = _lane_bcast(pu16, r)
                pv = zero
                for kk in range(_D // _L):
                    lo = pl.ds(kk * _L, _L)
                    hi = pl.ds(_D + kk * _L, _L)
                    ul, uh = ru[i, lo], ru[i, hi]
                    u = ul + fu * (uh - ul)
                    p = rp[i, lo]
                    n = rn[i, lo]
                    pv = pv + u * (p - n)
                    sacc = sacc + u * u + p * p + n * n
                parts[pl.ds((c * _CH + i) * _L, _L)] = pv
        return sacc

    sacc = lax.fori_loop(0, _NRND, round_body, zero)

    racc[...] = sacc
    pltpu.sync_copy(parts, xp_out.at[pl.ds(base * _L, _BPW * _L)])
    pltpu.sync_copy(racc, reg_out.at[pl.ds(wid * _L, _L)])


_sc_gather_dot = functools.partial(
    pl.kernel,
    mesh=plsc.VectorSubcoreMesh(core_axis_name="c", subcore_axis_name="s"),
    out_type=[
        jax.ShapeDtypeStruct((_BATCH * _L,), jnp.float32),
        jax.ShapeDtypeStruct((_NW * _L,), jnp.float32),
    ],
    scratch_types=[
        pltpu.VMEM((_NCHUNK, _CHUNK), jnp.int32),
        pltpu.VMEM((_BPW,), jnp.float32),
        pltpu.VMEM((_NCHUNK, _CHUNK), jnp.int32),
        pltpu.VMEM((_NCHUNK, _CHUNK), jnp.int32),
        pltpu.VMEM((_CH, 2 * _D), jnp.float32),
        pltpu.VMEM((_CH, _D), jnp.float32),
        pltpu.VMEM((_CH, _D), jnp.float32),
        pltpu.VMEM((_BPW * _L,), jnp.float32),
        pltpu.VMEM((_L,), jnp.float32),
        pltpu.SemaphoreType.DMA,
    ],
)(_sc_body)


_NSB = pl.cdiv(_N_TBL, 2 * _TS)


def _transpose_body(lo_ref, hi_ref, out_ref):
    out_ref[:, : _D] = lo_ref[...].T
    out_ref[:, _D :] = hi_ref[...].T


_transpose = pl.pallas_call(
    _transpose_body,
    grid=(_NSB,),
    in_specs=[
        pl.BlockSpec((_D, _TS), lambda g: (0, 2 * g)),
        pl.BlockSpec(
            (_D, _TS),
            lambda g: (0, jnp.minimum(2 * g + 1, _N_TBL // _TS - 1)),
        ),
    ],
    out_specs=pl.BlockSpec((_TS, 2 * _D), lambda g: (g, 0)),
    out_shape=jax.ShapeDtypeStruct((_NSB * _TS, 2 * _D), jnp.float32),
)


def _finish_body(xp_ref, regp_ref, loss_ref, reg_ref):
    xp = xp_ref[...]
    grp = lax.broadcasted_iota(jnp.int32, (128, 8), 0) // _L
    col = lax.broadcasted_iota(jnp.int32, (128, 8), 1)
    diff = (grp - col).astype(jnp.float32)
    sel = 1.0 - jnp.abs(jnp.sign(diff))
    x = lax.dot_general(xp, sel, (((1,), (0,)), ((), ())),
                        preferred_element_type=jnp.float32)
    ls = jnp.minimum(x, 0.0) - jnp.log1p(jnp.exp(-jnp.abs(x)))
    loss_ref[...] = jnp.reshape(-jnp.sum(ls) * (1.0 / _BATCH), (1, 1))
    reg_ref[...] = jnp.reshape(jnp.sum(regp_ref[...]) * (1.0 / _BATCH), (1, 1))


_finish = pl.pallas_call(
    _finish_body,
    out_shape=(
        jax.ShapeDtypeStruct((1, 1), jnp.float32),
        jax.ShapeDtypeStruct((1, 1), jnp.float32),
    ),
)


def kernel(users, pos, neg, user_table, item_table):
    ut_t = user_table.T
    ut2 = _transpose(ut_t, ut_t)
    it3 = item_table.reshape(-1, _TPB, _D)
    xp, regp = _sc_gather_dot(users, pos, neg, ut2, it3)
    loss, reg = _finish(xp.reshape(_BATCH // 8, 128), regp.reshape(4, 128))
    return loss.reshape(()), reg.reshape(())
